# baseline (speedup 1.0000x reference)
import functools
import math

import jax
import jax.numpy as jnp
from jax import lax
from jax.experimental import pallas as pl
from jax.experimental.pallas import tpu as pltpu
from jax.experimental.pallas import tpu_sc as plsc

_VOCAB = 50368
_HIDDEN = 768
_B, _S = 4, 8192
_EPS = 1e-12
_SCALE = math.sqrt(float(_HIDDEN))
_L = 16
_NJ = _HIDDEN // _L

_info = plsc.get_sparse_core_info()
_NC, _NS = _info.num_cores, _info.num_subcores
_NW = _NC * _NS
_SPW = _S // _NW
_SB = 64
_NSB = _SPW // _SB

_mesh = plsc.VectorSubcoreMesh(core_axis_name="c", subcore_axis_name="s")


def _rsqrt(x):
    i = lax.bitcast_convert_type(x, jnp.int32)
    i = jnp.int32(0x5F3759DF) - lax.shift_right_logical(i, 1)
    y = lax.bitcast_convert_type(i, jnp.float32)
    for _ in range(3):
        y = y * (1.5 - 0.5 * x * y * y)
    return y


@functools.partial(
    pl.kernel,
    out_type=jax.ShapeDtypeStruct((_B * _S, _HIDDEN), jnp.float32),
    mesh=_mesh,
    compiler_params=pltpu.CompilerParams(needs_layout_passes=False),
    scratch_types=[
        pltpu.VMEM((_SB,), jnp.int32),
        pltpu.VMEM((_SB, _HIDDEN), jnp.float32),
        pltpu.VMEM((_SB, _HIDDEN), jnp.float32),
        pltpu.VMEM((1, _HIDDEN), jnp.float32),
        pltpu.VMEM((_HIDDEN,), jnp.float32),
        pltpu.VMEM((_HIDDEN,), jnp.float32),
        pltpu.SemaphoreType.DMA,
    ],
)
def _sc_embed(ids_hbm, word_hbm, pos_hbm, tt_hbm, g_hbm, bt_hbm, out_hbm,
              ids_v, pos_v, rows_v, tt_v, g_v, bt_v, sem):
    wid = lax.axis_index("s") * _NC + lax.axis_index("c")
    s_base = wid * _SPW

    pltpu.sync_copy(tt_hbm.at[pl.ds(0, 1)], tt_v)
    pltpu.sync_copy(g_hbm, g_v)
    pltpu.sync_copy(bt_hbm, bt_v)

    def sblock(sb, _):
        s0 = s_base + sb * _SB
        pltpu.sync_copy(pos_hbm.at[pl.ds(s0, _SB)], pos_v)

        def comb_t(t, _):
            def comb_j(j, _):
                col = j * _L
                pos_v[t, pl.ds(col, _L)] = (
                    pos_v[t, pl.ds(col, _L)] + tt_v[0, pl.ds(col, _L)]
                )
                return 0
            return lax.fori_loop(0, _NJ, comb_j, 0)
        lax.fori_loop(0, _SB, comb_t, 0)

        def batch(b, _):
            tok0 = b * _S + s0
            pltpu.sync_copy(ids_hbm.at[pl.ds(tok0, _SB)], ids_v)
            pltpu.async_copy(word_hbm.at[ids_v], rows_v, sem).wait()

            def token(t, _):
                def p1(j, carry):
                    acc, acc2 = carry
                    col = j * _L
                    v = rows_v[t, pl.ds(col, _L)] * _SCALE + pos_v[t, pl.ds(col, _L)]
                    rows_v[t, pl.ds(col, _L)] = v
                    return acc + v, acc2 + v * v

                zero = jnp.zeros((_L,), jnp.float32)
                acc, acc2 = lax.fori_loop(0, _NJ, p1, (zero, zero))
                s1 = jnp.sum(acc)
                s2 = jnp.sum(acc2)
                mean = s1 * (1.0 / _HIDDEN)
                var = s2 * (1.0 / _HIDDEN) - mean * mean
                inv = _rsqrt(var + _EPS)
                inv_v = jnp.broadcast_to(inv, (_L,))
                nb_v = jnp.broadcast_to(-mean * inv, (_L,))

                def p2(j, _):
                    col = j * _L
                    x = rows_v[t, pl.ds(col, _L)]
                    y = x * inv_v + nb_v
                    rows_v[t, pl.ds(col, _L)] = (
                        y * g_v[pl.ds(col, _L)] + bt_v[pl.ds(col, _L)]
                    )
                    return 0
                return lax.fori_loop(0, _NJ, p2, 0)

            lax.fori_loop(0, _SB, token, 0)
            pltpu.sync_copy(rows_v, out_hbm.at[pl.ds(tok0, _SB)])
            return 0

        return lax.fori_loop(0, _B, batch, 0)

    lax.fori_loop(0, _NSB, sblock, 0)


def kernel(input_ids, word_table, pos_table, tt_table, ln_gamma, ln_beta):
    ids_flat = input_ids.reshape(-1)
    out_flat = _sc_embed(ids_flat, word_table, pos_table, tt_table,
                         ln_gamma, ln_beta)
    return out_flat.reshape(_B, _S, _HIDDEN)
